# baseline (speedup 1.0000x reference)
import functools

import jax
import jax.numpy as jnp
from jax import lax
from jax.experimental import pallas as pl
from jax.experimental.pallas import tpu as pltpu
from jax.experimental.pallas import tpu_sc as plsc

N = 50000
D = 64
DH = 32
E = 800000
NC = 2
NS = 16
CH = 256
SUB = 128
NSUB = CH // SUB
STEPS = 198
EP = 200 * CH * NS
NPAD = N + 8
RB = 128
NFULL = N // RB
NREM = N - NFULL * RB
ZFULL = NPAD // RB
ZREM = NPAD - ZFULL * RB


def _sc_aggregate(hpad, tpad, entcat, sheadp, srel16):
    mesh = plsc.VectorSubcoreMesh(
        core_axis_name="c", subcore_axis_name="s", num_cores=NC, num_subcores=NS
    )

    dbl = lambda t: [t, t]
    @functools.partial(
        pl.kernel,
        out_type=(jax.ShapeDtypeStruct((N, DH), jnp.float32),
                  jax.ShapeDtypeStruct((N, DH), jnp.float32)),
        mesh=mesh,
        compiler_params=pltpu.CompilerParams(use_tc_tiling_on_sc=False),
        scratch_types=[
            pltpu.VMEM_SHARED((NPAD, DH), jnp.float32),
            *[pltpu.VMEM((NSUB, SUB), jnp.int32)] * 3,
            *[pltpu.VMEM((CH,), jnp.int32)] * 3,
            *[pltpu.VMEM((CH,), jnp.int32)] * 3,
            *[pltpu.VMEM((CH,), jnp.float32)] * 3,
            *[pltpu.VMEM((CH, DH), jnp.float32)] * 3,
            pltpu.VMEM((16,), jnp.float32),
            *[pltpu.SemaphoreType.DMA] * 9,
        ],
    )
    def body(hpad_h, tpad_h, ent_h, shead_h, srel_h, out0_h, out1_h,
             acc, hidx0, hidx1, hidx2, tidx0, tidx1, tidx2,
             typ0, typ1, typ2, sh0, sh1, sh2, rows0, rows1, rows2, srl_v,
             semA0, semA1, semA2, semB0, semB1, semB2,
             semC0, semC1, semC2):
        rows = rows0
        c = lax.axis_index("c")
        s = lax.axis_index("s")
        zero16 = jnp.zeros((16,), jnp.float32)

        def zrow(i, _):
            rows[i, pl.ds(0, 16)] = zero16
            rows[i, pl.ds(16, 16)] = zero16
            return 0
        lax.fori_loop(0, RB, zrow, 0, unroll=4)

        ztrips = jnp.where(s < ZFULL % NS, ZFULL // NS + 1, ZFULL // NS)

        def zacc(i, _):
            blk = i * NS + s
            pltpu.sync_copy(rows.at[pl.ds(0, RB)],
                            acc.at[pl.ds(blk * RB, RB)])
            return 0
        lax.fori_loop(0, ztrips, zacc, 0)

        @pl.when(s == ZFULL % NS)
        def _():
            pltpu.sync_copy(rows.at[pl.ds(0, ZREM)],
                            acc.at[pl.ds(ZFULL * RB, ZREM)])

        pltpu.sync_copy(srel_h, srl_v)
        srl = srl_v[pl.ds(0, 16)]
        srel0 = srl[0]
        srel1 = srl[1]

        plsc.subcore_barrier()

        cN = c * N

        def fire_lin(off, hidxb, tidxb, sem):
            ds = []
            for j in range(NSUB):
                ds.append(pltpu.async_copy(
                    hpad_h.at[pl.ds(off + SUB * j, SUB)], hidxb.at[j], sem))
            ds.append(pltpu.async_copy(tpad_h.at[pl.ds(off, CH)], tidxb, sem))
            return ds

        def adj(tidxb, typb):
            def toff(i, _):
                p = tidxb[pl.ds(16 * i, 16)]
                typb[pl.ds(16 * i, 16)] = p >> 30
                tidxb[pl.ds(16 * i, 16)] = (p & 0x3FFFFFFF) + cN
                return 0
            lax.fori_loop(0, CH // 16, toff, 0, unroll=4)

        def fire_gath(hidxb, tidxb, shb, rowsb, sem):
            ds = []
            for j in range(NSUB):
                ds.append(pltpu.async_copy(
                    shead_h.at[hidxb.at[j]], shb.at[pl.ds(SUB * j, SUB)], sem))
                ds.append(pltpu.async_copy(
                    ent_h.at[tidxb.at[pl.ds(SUB * j, SUB)]],
                    rowsb.at[pl.ds(SUB * j, SUB)], sem))
            return ds

        def compute(shb, typb, rowsb):
            def attg(i, _):
                x = shb[pl.ds(16 * i, 16)]
                t16 = typb[pl.ds(16 * i, 16)]
                r = jnp.where(t16 == 0, srel0, srel1)
                a = 1.0 / (1.0 + jnp.exp(-(x + r)))
                shb[pl.ds(16 * i, 16)] = a
                return 0
            lax.fori_loop(0, CH // 16, attg, 0, unroll=2)

            def scale(g, _):
                a16 = shb[pl.ds(16 * g, 16)]
                for j in range(16):
                    i = 16 * g + j
                    a = a16[j]
                    rowsb[i, pl.ds(0, 16)] = rowsb[i, pl.ds(0, 16)] * a
                    rowsb[i, pl.ds(16, 16)] = rowsb[i, pl.ds(16, 16)] * a
                return 0
            lax.fori_loop(0, CH // 16, scale, 0)

        def fire_scat(rowsb, hidxb, sem):
            return [pltpu.async_copy(rowsb.at[pl.ds(SUB * j, SUB)],
                                     acc.at[hidxb.at[j]], sem, add=True)
                    for j in range(NSUB)]

        def off_of(k):
            return (k * NS + s) * CH

        def wait_lin_r(hidxb, tidxb, sem):
            for j in range(NSUB):
                pltpu.make_async_copy(hpad_h.at[pl.ds(0, SUB)],
                                      hidxb.at[j], sem).wait()
            pltpu.make_async_copy(tpad_h.at[pl.ds(0, CH)], tidxb, sem).wait()

        def wait_gath_r(hidxb, tidxb, shb, rowsb, sem):
            for j in range(NSUB):
                pltpu.make_async_copy(shead_h.at[hidxb.at[j]],
                                      shb.at[pl.ds(SUB * j, SUB)], sem).wait()
                pltpu.make_async_copy(ent_h.at[tidxb.at[pl.ds(SUB * j, SUB)]],
                                      rowsb.at[pl.ds(SUB * j, SUB)],
                                      sem).wait()

        def wait_scat_r(rowsb, hidxb, sem):
            for j in range(NSUB):
                pltpu.make_async_copy(rowsb.at[pl.ds(SUB * j, SUB)],
                                      acc.at[hidxb.at[j]], sem).wait()

        dl = fire_lin(off_of(0), hidx0, tidx0, semA0)
        for d in dl:
            d.wait()
        fire_lin(off_of(1), hidx1, tidx1, semA1)
        adj(tidx0, typ0)
        fire_gath(hidx0, tidx0, sh0, rows0, semB0)

        def step(m, _):
            ka = 3 * m

            @pl.when(m > 0)
            def _():
                wait_scat_r(rows2, hidx2, semC2)
            fire_lin(off_of(ka + 2), hidx2, tidx2, semA2)
            wait_lin_r(hidx1, tidx1, semA1)
            adj(tidx1, typ1)
            fire_gath(hidx1, tidx1, sh1, rows1, semB1)
            wait_gath_r(hidx0, tidx0, sh0, rows0, semB0)
            compute(sh0, typ0, rows0)
            dsa = fire_scat(rows0, hidx0, semC0)
            wait_lin_r(hidx2, tidx2, semA2)
            adj(tidx2, typ2)
            fire_gath(hidx2, tidx2, sh2, rows2, semB2)
            wait_gath_r(hidx1, tidx1, sh1, rows1, semB1)
            compute(sh1, typ1, rows1)
            dsb = fire_scat(rows1, hidx1, semC1)
            for d in dsa:
                d.wait()
            dl2 = fire_lin(off_of(ka + 3), hidx0, tidx0, semA0)
            for d in dl2:
                d.wait()
            adj(tidx0, typ0)
            fire_gath(hidx0, tidx0, sh0, rows0, semB0)
            wait_gath_r(hidx2, tidx2, sh2, rows2, semB2)
            compute(sh2, typ2, rows2)
            fire_scat(rows2, hidx2, semC2)
            for d in dsb:
                d.wait()
            fire_lin(off_of(ka + 4), hidx1, tidx1, semA1)
            return 0

        lax.fori_loop(0, STEPS // 3, step, 0)

        wait_scat_r(rows2, hidx2, semC2)
        wait_gath_r(hidx0, tidx0, sh0, rows0, semB0)
        wait_lin_r(hidx1, tidx1, semA1)

        plsc.subcore_barrier()

        wtrips = jnp.where(s < NFULL % NS, NFULL // NS + 1, NFULL // NS)

        def wrout(out_h):
            def wout(i, _):
                blk = i * NS + s
                pltpu.sync_copy(acc.at[pl.ds(blk * RB, RB)],
                                out_h.at[pl.ds(blk * RB, RB)])
                return 0
            lax.fori_loop(0, wtrips, wout, 0)

            @pl.when(s == NFULL % NS)
            def _():
                pltpu.sync_copy(acc.at[pl.ds(NFULL * RB, NREM)],
                                out_h.at[pl.ds(NFULL * RB, NREM)])

        @pl.when(c == 0)
        def _():
            wrout(out0_h)

        @pl.when(c == 1)
        def _():
            wrout(out1_h)

    return body(hpad, tpad, entcat, sheadp, srel16)


def _shead_body(ent_ref, rel_ref, wh_ref, wr_ref, ab_ref, o1_ref, o2_ref):
    o1_ref[...] = jnp.dot(ent_ref[...], wh_ref[...],
                          preferred_element_type=jnp.float32)
    o2_ref[...] = jnp.dot(rel_ref[...], wr_ref[...],
                          preferred_element_type=jnp.float32) + ab_ref[...]


def _shead(entity_emb, relation_emb, wh, wr, ab):
    return pl.pallas_call(
        _shead_body,
        grid=(25,),
        in_specs=[
            pl.BlockSpec((N // 25, D), lambda i: (i, 0)),
            pl.BlockSpec((2, D), lambda i: (0, 0)),
            pl.BlockSpec((D, 1), lambda i: (0, 0)),
            pl.BlockSpec((D, 1), lambda i: (0, 0)),
            pl.BlockSpec((1, 1), lambda i: (0, 0)),
        ],
        out_specs=(pl.BlockSpec((N // 25, 1), lambda i: (i, 0)),
                   pl.BlockSpec((2, 1), lambda i: (0, 0))),
        out_shape=(jax.ShapeDtypeStruct((N, 1), jnp.float32),
                   jax.ShapeDtypeStruct((2, 1), jnp.float32)),
    )(entity_emb, relation_emb, wh, wr, ab)


def _final_body(a0_ref, a1_ref, w_ref, b_ref, o_ref):
    w = w_ref[...]
    dn = (((1,), (1,)), ((), ()))
    acc = lax.dot_general(a0_ref[...], w[:, :DH], dn,
                          preferred_element_type=jnp.float32)
    acc = acc + lax.dot_general(a1_ref[...], w[:, DH:], dn,
                                preferred_element_type=jnp.float32)
    o_ref[...] = jnp.tanh(acc + b_ref[...])


def _final(a0, a1, W_w, b2d):
    return pl.pallas_call(
        _final_body,
        grid=(25,),
        in_specs=[
            pl.BlockSpec((2000, DH), lambda i: (i, 0)),
            pl.BlockSpec((2000, DH), lambda i: (i, 0)),
            pl.BlockSpec((D, D), lambda i: (0, 0)),
            pl.BlockSpec((1, D), lambda i: (0, 0)),
        ],
        out_specs=pl.BlockSpec((2000, D), lambda i: (i, 0)),
        out_shape=jax.ShapeDtypeStruct((N, D), jnp.float32),
    )(a0, a1, W_w, b2d)


def kernel(edge_index, edge_type, entity_emb, relation_emb, attn_w, attn_b,
           W_w, W_b):
    pad = EP - E
    i32 = jnp.int32
    hpad = jnp.concatenate([edge_index[0], jnp.full((pad,), N, i32)])
    tpack = edge_index[1] | (edge_type << 30)
    tpad = jnp.concatenate([tpack, jnp.zeros((pad,), i32)])
    entcat = jnp.concatenate([entity_emb[:, :DH], entity_emb[:, DH:]], axis=0)
    shead2, srel2 = _shead(entity_emb, relation_emb,
                           attn_w[0, :D].reshape(D, 1),
                           attn_w[0, D:].reshape(D, 1),
                           attn_b.reshape(1, 1))
    sheadp = jnp.concatenate([shead2[:, 0], jnp.zeros((8,), jnp.float32)])
    srel16 = jnp.concatenate([srel2[:, 0], jnp.zeros((14,), jnp.float32)])
    a0, a1 = _sc_aggregate(hpad, tpad, entcat, sheadp, srel16)
    return _final(a0, a1, W_w, W_b.reshape(1, D))


# speedup vs baseline: 1.0533x; 1.0533x over previous
import functools

import jax
import jax.numpy as jnp
from jax import lax
from jax.experimental import pallas as pl
from jax.experimental.pallas import tpu as pltpu
from jax.experimental.pallas import tpu_sc as plsc

N = 50000
D = 64
DH = 32
E = 800000
NC = 2
NS = 16
CH = 256
SUB = 128
NSUB = CH // SUB
STEPS = -(-E // (CH * NS))
EP = STEPS * CH * NS
NPAD = N + 8
RB = 128
NFULL = N // RB
NREM = N - NFULL * RB
ZFULL = NPAD // RB
ZREM = NPAD - ZFULL * RB
NOUT = 50176


def _sc_aggregate(hpad, tpad, entcat, sheadp, srel16):
    mesh = plsc.VectorSubcoreMesh(
        core_axis_name="c", subcore_axis_name="s", num_cores=NC, num_subcores=NS
    )

    dbl = lambda t: [t, t]
    @functools.partial(
        pl.kernel,
        out_type=(jax.ShapeDtypeStruct((NOUT, DH), jnp.float32),
                  jax.ShapeDtypeStruct((NOUT, DH), jnp.float32)),
        mesh=mesh,
        compiler_params=pltpu.CompilerParams(use_tc_tiling_on_sc=False),
        scratch_types=[
            pltpu.VMEM_SHARED((NPAD, DH), jnp.float32),
            *dbl(pltpu.VMEM((NSUB, SUB), jnp.int32)),
            *dbl(pltpu.VMEM((CH,), jnp.int32)),
            *dbl(pltpu.VMEM((CH,), jnp.int32)),
            *dbl(pltpu.VMEM((CH,), jnp.float32)),
            *dbl(pltpu.VMEM((CH, DH), jnp.float32)),
            pltpu.VMEM((16,), jnp.float32),
            *[pltpu.SemaphoreType.DMA] * 6,
        ],
    )
    def body(hpad_h, tpad_h, ent_h, shead_h, srel_h, out0_h, out1_h,
             acc, hidx0, hidx1, tidx0, tidx1, typ0, typ1, sh0, sh1,
             rows0, rows1, srl_v,
             semA0, semA1, semB0, semB1, semC0, semC1):
        rows = rows0
        c = lax.axis_index("c")
        s = lax.axis_index("s")
        zero16 = jnp.zeros((16,), jnp.float32)

        def zrow(i, _):
            rows[i, pl.ds(0, 16)] = zero16
            rows[i, pl.ds(16, 16)] = zero16
            return 0
        lax.fori_loop(0, RB, zrow, 0, unroll=4)

        ztrips = jnp.where(s < ZFULL % NS, ZFULL // NS + 1, ZFULL // NS)

        def zacc(i, _):
            blk = i * NS + s
            pltpu.sync_copy(rows.at[pl.ds(0, RB)],
                            acc.at[pl.ds(blk * RB, RB)])
            return 0
        lax.fori_loop(0, ztrips, zacc, 0)

        @pl.when(s == ZFULL % NS)
        def _():
            pltpu.sync_copy(rows.at[pl.ds(0, ZREM)],
                            acc.at[pl.ds(ZFULL * RB, ZREM)])

        pltpu.sync_copy(srel_h, srl_v)
        srl = srl_v[pl.ds(0, 16)]
        srel0 = srl[0]
        srel1 = srl[1]

        plsc.subcore_barrier()

        cN = c * N

        def fire_lin(off, hidxb, tidxb, sem):
            ds = []
            for j in range(NSUB):
                ds.append(pltpu.async_copy(
                    hpad_h.at[pl.ds(off + SUB * j, SUB)], hidxb.at[j], sem))
            ds.append(pltpu.async_copy(tpad_h.at[pl.ds(off, CH)], tidxb, sem))
            return ds

        def adj(tidxb, typb):
            def toff(i, _):
                p = tidxb[pl.ds(16 * i, 16)]
                typb[pl.ds(16 * i, 16)] = p >> 30
                tidxb[pl.ds(16 * i, 16)] = (p & 0x3FFFFFFF) + cN
                return 0
            lax.fori_loop(0, CH // 16, toff, 0, unroll=4)

        def fire_gath(hidxb, tidxb, shb, rowsb, sem):
            ds = []
            for j in range(NSUB):
                ds.append(pltpu.async_copy(
                    shead_h.at[hidxb.at[j]], shb.at[pl.ds(SUB * j, SUB)], sem))
                ds.append(pltpu.async_copy(
                    ent_h.at[tidxb.at[pl.ds(SUB * j, SUB)]],
                    rowsb.at[pl.ds(SUB * j, SUB)], sem))
            return ds

        def compute(shb, typb, rowsb):
            def attg(i, _):
                x = shb[pl.ds(16 * i, 16)]
                t16 = typb[pl.ds(16 * i, 16)]
                r = jnp.where(t16 == 0, srel0, srel1)
                a = 1.0 / (1.0 + jnp.exp(-(x + r)))
                shb[pl.ds(16 * i, 16)] = a
                return 0
            lax.fori_loop(0, CH // 16, attg, 0, unroll=2)

            def scale(g, _):
                a16 = shb[pl.ds(16 * g, 16)]
                for j in range(16):
                    i = 16 * g + j
                    a = a16[j]
                    rowsb[i, pl.ds(0, 16)] = rowsb[i, pl.ds(0, 16)] * a
                    rowsb[i, pl.ds(16, 16)] = rowsb[i, pl.ds(16, 16)] * a
                return 0
            lax.fori_loop(0, CH // 16, scale, 0)

        def fire_scat(rowsb, hidxb, sem):
            return [pltpu.async_copy(rowsb.at[pl.ds(SUB * j, SUB)],
                                     acc.at[hidxb.at[j]], sem, add=True)
                    for j in range(NSUB)]

        def step(m, _):
            offa = ((2 * m) * NS + s) * CH
            offb = ((2 * m + 1) * NS + s) * CH
            dla = fire_lin(offa, hidx0, tidx0, semA0)
            dlb = fire_lin(offb, hidx1, tidx1, semA1)
            for d in dla:
                d.wait()
            adj(tidx0, typ0)
            dga = fire_gath(hidx0, tidx0, sh0, rows0, semB0)
            for d in dlb:
                d.wait()
            adj(tidx1, typ1)
            dgb = fire_gath(hidx1, tidx1, sh1, rows1, semB1)
            for d in dga:
                d.wait()
            compute(sh0, typ0, rows0)
            dsa = fire_scat(rows0, hidx0, semC0)
            for d in dgb:
                d.wait()
            compute(sh1, typ1, rows1)
            dsb = fire_scat(rows1, hidx1, semC1)
            for d in dsa:
                d.wait()
            for d in dsb:
                d.wait()
            return 0

        lax.fori_loop(0, STEPS // 2, step, 0)

        plsc.subcore_barrier()

        wtrips = jnp.where(s < NFULL % NS, NFULL // NS + 1, NFULL // NS)

        def wrout(out_h):
            def wout(i, _):
                blk = i * NS + s
                pltpu.sync_copy(acc.at[pl.ds(blk * RB, RB)],
                                out_h.at[pl.ds(blk * RB, RB)])
                return 0
            lax.fori_loop(0, wtrips, wout, 0)

            @pl.when(s == NFULL % NS)
            def _():
                pltpu.sync_copy(acc.at[pl.ds(NFULL * RB, NREM)],
                                out_h.at[pl.ds(NFULL * RB, NREM)])

        @pl.when(c == 0)
        def _():
            wrout(out0_h)

        @pl.when(c == 1)
        def _():
            wrout(out1_h)

    return body(hpad, tpad, entcat, sheadp, srel16)


def _shead_body(ent_ref, rel_ref, wh_ref, wr_ref, ab_ref, o1_ref, o2_ref):
    o1_ref[...] = jnp.dot(ent_ref[...], wh_ref[...],
                          preferred_element_type=jnp.float32)
    o2_ref[...] = jnp.dot(rel_ref[...], wr_ref[...],
                          preferred_element_type=jnp.float32) + ab_ref[...]


def _shead(entity_emb, relation_emb, wh, wr, ab):
    return pl.pallas_call(
        _shead_body,
        grid=(25,),
        in_specs=[
            pl.BlockSpec((N // 25, D), lambda i: (i, 0)),
            pl.BlockSpec((2, D), lambda i: (0, 0)),
            pl.BlockSpec((D, 1), lambda i: (0, 0)),
            pl.BlockSpec((D, 1), lambda i: (0, 0)),
            pl.BlockSpec((1, 1), lambda i: (0, 0)),
        ],
        out_specs=(pl.BlockSpec((N // 25, 1), lambda i: (i, 0)),
                   pl.BlockSpec((2, 1), lambda i: (0, 0))),
        out_shape=(jax.ShapeDtypeStruct((N, 1), jnp.float32),
                   jax.ShapeDtypeStruct((2, 1), jnp.float32)),
    )(entity_emb, relation_emb, wh, wr, ab)


def _final_body(a0_ref, a1_ref, m0_ref, m1_ref, b_ref, o_ref):
    acc = jnp.dot(a0_ref[...], m0_ref[...], preferred_element_type=jnp.float32)
    acc = acc + jnp.dot(a1_ref[...], m1_ref[...],
                        preferred_element_type=jnp.float32)
    o_ref[...] = jnp.tanh(acc + b_ref[...])


def _final(a0r, a1r, M0, M1, b4):
    return pl.pallas_call(
        _final_body,
        grid=(14,),
        in_specs=[
            pl.BlockSpec((896, 4 * DH), lambda i: (i, 0)),
            pl.BlockSpec((896, 4 * DH), lambda i: (i, 0)),
            pl.BlockSpec((4 * DH, 4 * D), lambda i: (0, 0)),
            pl.BlockSpec((4 * DH, 4 * D), lambda i: (0, 0)),
            pl.BlockSpec((1, 4 * D), lambda i: (0, 0)),
        ],
        out_specs=pl.BlockSpec((896, 4 * D), lambda i: (i, 0)),
        out_shape=jax.ShapeDtypeStruct((NOUT // 4, 4 * D), jnp.float32),
    )(a0r, a1r, M0, M1, b4)


def kernel(edge_index, edge_type, entity_emb, relation_emb, attn_w, attn_b,
           W_w, W_b):
    pad = EP - E
    i32 = jnp.int32
    hpad = jnp.concatenate([edge_index[0], jnp.full((pad,), N, i32)])
    tpack = edge_index[1] | (edge_type << 30)
    tpad = jnp.concatenate([tpack, jnp.zeros((pad,), i32)])
    entcat = jnp.concatenate([entity_emb[:, :DH], entity_emb[:, DH:]], axis=0)
    shead2, srel2 = _shead(entity_emb, relation_emb,
                           attn_w[0, :D].reshape(D, 1),
                           attn_w[0, D:].reshape(D, 1),
                           attn_b.reshape(1, 1))
    sheadp = jnp.concatenate([shead2[:, 0], jnp.zeros((8,), jnp.float32)])
    srel16 = jnp.concatenate([srel2[:, 0], jnp.zeros((14,), jnp.float32)])
    a0, a1 = _sc_aggregate(hpad, tpad, entcat, sheadp, srel16)
    from jax.scipy.linalg import block_diag
    M0 = block_diag(*([W_w[:, :DH].T] * 4))
    M1 = block_diag(*([W_w[:, DH:].T] * 4))
    b4 = jnp.tile(W_b, 4).reshape(1, 4 * D)
    out4 = _final(a0.reshape(NOUT // 4, 4 * DH), a1.reshape(NOUT // 4, 4 * DH),
                  M0, M1, b4)
    return out4[:N // 4].reshape(N, D)


# speedup vs baseline: 1.0876x; 1.0325x over previous
import functools

import jax
import jax.numpy as jnp
from jax import lax
from jax.experimental import pallas as pl
from jax.experimental.pallas import tpu as pltpu
from jax.experimental.pallas import tpu_sc as plsc

N = 50000
D = 64
DH = 32
E = 800000
NC = 2
NS = 16
CH = 256
SUB = 128
NSUB = CH // SUB
STEPS = -(-E // (CH * NS))
EP = STEPS * CH * NS
NPAD = N + 8
RB = 128
NFULL = N // RB
NREM = N - NFULL * RB
ZFULL = NPAD // RB
ZREM = NPAD - ZFULL * RB
NOUT = 50176


def _sc_aggregate(hpad, tpad, entcat, sheadp, srel16):
    mesh = plsc.VectorSubcoreMesh(
        core_axis_name="c", subcore_axis_name="s", num_cores=NC, num_subcores=NS
    )

    dbl = lambda t: [t, t]
    @functools.partial(
        pl.kernel,
        out_type=(jax.ShapeDtypeStruct((NOUT, DH), jnp.float32),
                  jax.ShapeDtypeStruct((NOUT, DH), jnp.float32)),
        mesh=mesh,
        compiler_params=pltpu.CompilerParams(use_tc_tiling_on_sc=False),
        scratch_types=[
            pltpu.VMEM_SHARED((NPAD, DH), jnp.float32),
            *dbl(pltpu.VMEM((NSUB, SUB), jnp.int32)),
            *dbl(pltpu.VMEM((CH,), jnp.int32)),
            *dbl(pltpu.VMEM((CH,), jnp.int32)),
            *dbl(pltpu.VMEM((CH,), jnp.float32)),
            *dbl(pltpu.VMEM((CH, DH), jnp.float32)),
            pltpu.VMEM((16,), jnp.float32),
            *[pltpu.SemaphoreType.DMA] * 6,
        ],
    )
    def body(hpad_h, tpad_h, ent_h, shead_h, srel_h, out0_h, out1_h,
             acc, hidx0, hidx1, tidx0, tidx1, typ0, typ1, sh0, sh1,
             rows0, rows1, srl_v,
             semA0, semA1, semB0, semB1, semC0, semC1):
        rows = rows0
        c = lax.axis_index("c")
        s = lax.axis_index("s")
        zero16 = jnp.zeros((16,), jnp.float32)

        def zrow(i, _):
            rows[i, pl.ds(0, 16)] = zero16
            rows[i, pl.ds(16, 16)] = zero16
            return 0
        lax.fori_loop(0, RB, zrow, 0, unroll=4)

        ztrips = jnp.where(s < ZFULL % NS, ZFULL // NS + 1, ZFULL // NS)

        def zacc(i, _):
            blk = i * NS + s
            pltpu.sync_copy(rows.at[pl.ds(0, RB)],
                            acc.at[pl.ds(blk * RB, RB)])
            return 0
        lax.fori_loop(0, ztrips, zacc, 0)

        @pl.when(s == ZFULL % NS)
        def _():
            pltpu.sync_copy(rows.at[pl.ds(0, ZREM)],
                            acc.at[pl.ds(ZFULL * RB, ZREM)])

        pltpu.sync_copy(srel_h, srl_v)
        srl = srl_v[pl.ds(0, 16)]
        srel0 = srl[0]
        srel1 = srl[1]

        plsc.subcore_barrier()

        cN = c * N

        def fire_lin(off, hidxb, tidxb, sem):
            ds = []
            for j in range(NSUB):
                ds.append(pltpu.async_copy(
                    hpad_h.at[pl.ds(off + SUB * j, SUB)], hidxb.at[j], sem))
            ds.append(pltpu.async_copy(tpad_h.at[pl.ds(off, CH)], tidxb, sem))
            return ds

        def adj(tidxb, typb):
            def toff(i, _):
                p = tidxb[pl.ds(16 * i, 16)]
                typb[pl.ds(16 * i, 16)] = p >> 30
                tidxb[pl.ds(16 * i, 16)] = (p & 0x3FFFFFFF) + cN
                return 0
            lax.fori_loop(0, CH // 16, toff, 0, unroll=4)

        def fire_gath(hidxb, tidxb, shb, rowsb, sem):
            ds = []
            for j in range(NSUB):
                ds.append(pltpu.async_copy(
                    shead_h.at[hidxb.at[j]], shb.at[pl.ds(SUB * j, SUB)], sem))
                ds.append(pltpu.async_copy(
                    ent_h.at[tidxb.at[pl.ds(SUB * j, SUB)]],
                    rowsb.at[pl.ds(SUB * j, SUB)], sem))
            return ds

        def compute(shb, typb, rowsb):
            def attg(i, _):
                x = shb[pl.ds(16 * i, 16)]
                t16 = typb[pl.ds(16 * i, 16)]
                r = jnp.where(t16 == 0, srel0, srel1)
                a = 1.0 / (1.0 + jnp.exp(-(x + r)))
                shb[pl.ds(16 * i, 16)] = a
                return 0
            lax.fori_loop(0, CH // 16, attg, 0, unroll=2)

            def scale(g, _):
                a16 = shb[pl.ds(16 * g, 16)]
                for j in range(16):
                    i = 16 * g + j
                    a = a16[j]
                    rowsb[i, pl.ds(0, 16)] = rowsb[i, pl.ds(0, 16)] * a
                    rowsb[i, pl.ds(16, 16)] = rowsb[i, pl.ds(16, 16)] * a
                return 0
            lax.fori_loop(0, CH // 16, scale, 0)

        def fire_scat(rowsb, hidxb, sem):
            return [pltpu.async_copy(rowsb.at[pl.ds(SUB * j, SUB)],
                                     acc.at[hidxb.at[j]], sem, add=True)
                    for j in range(NSUB)]

        def step(m, _):
            offa = ((2 * m) * NS + s) * CH
            offb = ((2 * m + 1) * NS + s) * CH
            dla = fire_lin(offa, hidx0, tidx0, semA0)
            dlb = fire_lin(offb, hidx1, tidx1, semA1)
            for d in dla:
                d.wait()
            adj(tidx0, typ0)
            dga = fire_gath(hidx0, tidx0, sh0, rows0, semB0)
            for d in dlb:
                d.wait()
            adj(tidx1, typ1)
            dgb = fire_gath(hidx1, tidx1, sh1, rows1, semB1)
            for d in dga:
                d.wait()
            compute(sh0, typ0, rows0)
            dsa = fire_scat(rows0, hidx0, semC0)
            for d in dgb:
                d.wait()
            compute(sh1, typ1, rows1)
            dsb = fire_scat(rows1, hidx1, semC1)
            for d in dsa:
                d.wait()
            for d in dsb:
                d.wait()
            return 0

        lax.fori_loop(0, STEPS // 2, step, 0)

        plsc.subcore_barrier()

        wtrips = jnp.where(s < NFULL % NS, NFULL // NS + 1, NFULL // NS)

        def wrout(out_h):
            def wout(i, _):
                blk = i * NS + s
                pltpu.sync_copy(acc.at[pl.ds(blk * RB, RB)],
                                out_h.at[pl.ds(blk * RB, RB)])
                return 0
            lax.fori_loop(0, wtrips, wout, 0)

            @pl.when(s == NFULL % NS)
            def _():
                pltpu.sync_copy(acc.at[pl.ds(NFULL * RB, NREM)],
                                out_h.at[pl.ds(NFULL * RB, NREM)])

        @pl.when(c == 0)
        def _():
            wrout(out0_h)

        @pl.when(c == 1)
        def _():
            wrout(out1_h)

    return body(hpad, tpad, entcat, sheadp, srel16)


def _shead_body(ent_ref, rel_ref, wh1_ref, wr_ref, ab_ref, o1_ref, o2_ref):
    wh1 = wh1_ref[...]
    dn = (((1,), (1,)), ((), ()))
    for r in range(56):
        o1_ref[pl.ds(r, 1), :] = lax.dot_general(
            wh1, ent_ref[pl.ds(128 * r, 128), :], dn,
            preferred_element_type=jnp.float32)
    o2_ref[...] = jnp.dot(rel_ref[...], wr_ref[...],
                          preferred_element_type=jnp.float32) + ab_ref[...]


def _shead(entity_pad, relation_emb, wh1, wr, ab):
    return pl.pallas_call(
        _shead_body,
        grid=(7,),
        in_specs=[
            pl.BlockSpec((NOUT // 7, D), lambda i: (i, 0)),
            pl.BlockSpec((2, D), lambda i: (0, 0)),
            pl.BlockSpec((1, D), lambda i: (0, 0)),
            pl.BlockSpec((D, 1), lambda i: (0, 0)),
            pl.BlockSpec((1, 1), lambda i: (0, 0)),
        ],
        out_specs=(pl.BlockSpec((56, 128), lambda i: (i, 0)),
                   pl.BlockSpec((2, 1), lambda i: (0, 0))),
        out_shape=(jax.ShapeDtypeStruct((NOUT // 128, 128), jnp.float32),
                   jax.ShapeDtypeStruct((2, 1), jnp.float32)),
    )(entity_pad, relation_emb, wh1, wr, ab)


def _final_body(a0_ref, a1_ref, m0_ref, m1_ref, b_ref, o_ref):
    acc = jnp.dot(a0_ref[...], m0_ref[...], preferred_element_type=jnp.float32)
    acc = acc + jnp.dot(a1_ref[...], m1_ref[...],
                        preferred_element_type=jnp.float32)
    o_ref[...] = jnp.tanh(acc + b_ref[...])


def _final(a0r, a1r, M0, M1, b4):
    return pl.pallas_call(
        _final_body,
        grid=(14,),
        in_specs=[
            pl.BlockSpec((896, 4 * DH), lambda i: (i, 0)),
            pl.BlockSpec((896, 4 * DH), lambda i: (i, 0)),
            pl.BlockSpec((4 * DH, 4 * D), lambda i: (0, 0)),
            pl.BlockSpec((4 * DH, 4 * D), lambda i: (0, 0)),
            pl.BlockSpec((1, 4 * D), lambda i: (0, 0)),
        ],
        out_specs=pl.BlockSpec((896, 4 * D), lambda i: (i, 0)),
        out_shape=jax.ShapeDtypeStruct((NOUT // 4, 4 * D), jnp.float32),
    )(a0r, a1r, M0, M1, b4)


def kernel(edge_index, edge_type, entity_emb, relation_emb, attn_w, attn_b,
           W_w, W_b):
    pad = EP - E
    i32 = jnp.int32
    hpad = jnp.concatenate([edge_index[0], jnp.full((pad,), N, i32)])
    tpack = edge_index[1] | (edge_type << 30)
    tpad = jnp.concatenate([tpack, jnp.zeros((pad,), i32)])
    entcat = jnp.concatenate([entity_emb[:, :DH], entity_emb[:, DH:]], axis=0)
    entity_pad = jnp.concatenate(
        [entity_emb, jnp.zeros((NOUT - N, D), jnp.float32)])
    shead2, srel2 = _shead(entity_pad, relation_emb,
                           attn_w[0, :D].reshape(1, D),
                           attn_w[0, D:].reshape(D, 1),
                           attn_b.reshape(1, 1))
    sheadp = shead2.reshape(NOUT)
    srel16 = jnp.concatenate([srel2[:, 0], jnp.zeros((14,), jnp.float32)])
    a0, a1 = _sc_aggregate(hpad, tpad, entcat, sheadp, srel16)
    from jax.scipy.linalg import block_diag
    M0 = block_diag(*([W_w[:, :DH].T] * 4))
    M1 = block_diag(*([W_w[:, DH:].T] * 4))
    b4 = jnp.tile(W_b, 4).reshape(1, 4 * D)
    out4 = _final(a0.reshape(NOUT // 4, 4 * DH), a1.reshape(NOUT // 4, 4 * DH),
                  M0, M1, b4)
    return out4[:N // 4].reshape(N, D)
